# Initial kernel scaffold; baseline (speedup 1.0000x reference)
import jax
import jax.numpy as jnp
from jax.experimental import pallas as pl


def kernel(cls_data, loc_data, landm_data, priors, targets):
    raise NotImplementedError("write your pallas kernel here")


# speedup vs baseline: 84.4963x; 84.4963x over previous
import functools

import jax
import jax.numpy as jnp
from jax import lax
from jax.experimental import pallas as pl

_B, _P, _G, _C = 32, 16800, 16, 2
_TH = 0.35
_NEGPOS = 7
_V0, _V1 = 0.1, 0.2


def _smooth_l1(d):
    ad = jnp.abs(d)
    return jnp.where(ad < 1.0, 0.5 * d * d, ad - 0.5)


def _phase1_body(cls_ref, loc_ref, landm_ref, pri_ref, tgt_ref,
                 v_ref, pos_ref, sums_ref):
    G, P = _G, _P
    pri = pri_ref[...]
    pcx, pcy = pri[0:1], pri[1:2]
    pw, ph = pri[2:3], pri[3:4]
    px1 = pcx - pw * 0.5
    py1 = pcy - ph * 0.5
    px2 = pcx + pw * 0.5
    py2 = pcy + ph * 0.5

    tgt = tgt_ref[0]
    tx1, ty1 = tgt[:, 0:1], tgt[:, 1:2]
    tx2, ty2 = tgt[:, 2:3], tgt[:, 3:4]

    iw = jnp.maximum(jnp.minimum(tx2, px2) - jnp.maximum(tx1, px1), 0.0)
    ih = jnp.maximum(jnp.minimum(ty2, py2) - jnp.maximum(ty1, py1), 0.0)
    inter = iw * ih
    area_t = (tx2 - tx1) * (ty2 - ty1)
    area_p = pw * ph
    ov = inter / (area_t + area_p - inter)

    gi = lax.broadcasted_iota(jnp.int32, (G, P), 0)
    pi = lax.broadcasted_iota(jnp.int32, (G, P), 1)

    bto = jnp.max(ov, axis=0, keepdims=True)
    bti = jnp.min(jnp.where(ov == bto, gi, G), axis=0, keepdims=True)
    bpo = jnp.max(ov, axis=1, keepdims=True)
    bpi = jnp.min(jnp.where(ov == bpo, pi, P), axis=1, keepdims=True)
    valid = bpo >= 0.2

    eqbp = bpi == pi
    gidx = jnp.max(jnp.where(eqbp, gi, -1), axis=0, keepdims=True)
    gval = jnp.max(jnp.where(eqbp & valid, gi, -1), axis=0, keepdims=True)
    has = gidx >= 0
    bto2 = jnp.where(has & (gval == gidx), 2.0, bto)
    bti2 = jnp.where(has, gidx, bti)

    onehot = (gi == bti2).astype(jnp.float32)
    m = lax.dot_general(tgt, onehot, (((0,), (0,)), ((), ())),
                        precision=lax.Precision.HIGHEST,
                        preferred_element_type=jnp.float32)

    label = m[14:15]
    conf = jnp.where(bto2 < _TH, 0, label.astype(jnp.int32))
    posf = (conf != 0).astype(jnp.float32)

    mx1, my1, mx2, my2 = m[0:1], m[1:2], m[2:3], m[3:4]
    gcx = ((mx1 + mx2) * 0.5 - pcx) / (_V0 * pw)
    gcy = ((my1 + my2) * 0.5 - pcy) / (_V0 * ph)
    gw = jnp.log((mx2 - mx1) / pw) * (1.0 / _V1)
    gh = jnp.log((my2 - my1) / ph) * (1.0 / _V1)
    gloc = jnp.concatenate([gcx, gcy, gw, gh], axis=0)
    loss_l = jnp.sum(_smooth_l1(loc_ref[0] - gloc) * posf)

    lm = m[4:14]
    pc10 = jnp.concatenate([pcx, pcy] * 5, axis=0)
    pwh10 = jnp.concatenate([pw, ph] * 5, axis=0)
    glm = (lm - pc10) / (_V0 * pwh10)
    loss_lm = jnp.sum(_smooth_l1(landm_ref[0] - glm) * posf)

    cls0, cls1 = cls_ref[0, 0:1], cls_ref[0, 1:2]
    mc = jnp.maximum(cls0, cls1)
    lse = mc + jnp.log(jnp.exp(cls0 - mc) + jnp.exp(cls1 - mc))
    csel = jnp.where(conf == 0, cls0, cls1)
    v = lse - csel

    num_pos = jnp.sum(posf)

    v_ref[0] = v
    pos_ref[0] = posf
    li = lax.broadcasted_iota(jnp.int32, (1, 128), 1)
    sums_ref[0] = jnp.where(
        li == 0, loss_l, jnp.where(li == 1, loss_lm,
                                   jnp.where(li == 2, num_pos, 0.0)))


def _phase2_body(v_ref, pos_ref, sums_ref, out_ref):
    B, P = _B, _P
    v = v_ref[...]
    posf = pos_ref[...]
    num_pos = sums_ref[:, 2:3]
    k = jnp.minimum(_NEGPOS * num_pos.astype(jnp.int32), P - 1)

    vbits = jnp.maximum(lax.bitcast_convert_type(v, jnp.int32), 0)

    def val_step(_, c):
        lo, hi = c
        mid = lo + lax.shift_right_logical(hi - lo + 1, 1)
        cnt = jnp.sum((vbits >= mid).astype(jnp.int32), axis=1, keepdims=True)
        ok = cnt >= k
        return jnp.where(ok, mid, lo), jnp.where(ok, hi, mid - 1)

    lo0 = jnp.zeros((B, 1), jnp.int32)
    hi0 = jnp.full((B, 1), 0x7F800000, jnp.int32)
    t, _ = lax.fori_loop(0, 31, val_step, (lo0, hi0))

    gt = vbits > t
    cnt_gt = jnp.sum(gt.astype(jnp.int32), axis=1, keepdims=True)
    r = k - cnt_gt
    tie = vbits == t
    idxs = lax.broadcasted_iota(jnp.int32, (B, P), 1)

    def idx_step(_, c):
        lo, hi = c
        mid = lax.shift_right_logical(lo + hi, 1)
        cnt = jnp.sum((tie & (idxs < mid)).astype(jnp.int32),
                      axis=1, keepdims=True)
        ok = cnt >= r
        return jnp.where(ok, lo, mid + 1), jnp.where(ok, mid, hi)

    lo0 = jnp.zeros((B, 1), jnp.int32)
    hi0 = jnp.full((B, 1), P, jnp.int32)
    _, cut = lax.fori_loop(0, 15, idx_step, (lo0, hi0))

    sel = jnp.maximum(posf, (gt | (tie & (idxs < cut))).astype(jnp.float32))
    loss_c = jnp.sum(v * sel)

    n = jnp.maximum(jnp.sum(num_pos), 1.0)
    loss_l = jnp.sum(sums_ref[:, 0:1]) / n
    loss_cf = loss_c / n
    loss_lm = jnp.sum(sums_ref[:, 1:2]) / n

    li = lax.broadcasted_iota(jnp.int32, (1, 128), 1)
    out_ref[...] = jnp.where(
        li == 0, loss_l, jnp.where(li == 1, loss_cf,
                                   jnp.where(li == 2, loss_lm, 0.0)))


@jax.jit
def kernel(cls_data, loc_data, landm_data, priors, targets):
    B, P = _B, _P
    cls_t = jnp.transpose(cls_data, (0, 2, 1))
    loc_t = jnp.transpose(loc_data, (0, 2, 1))
    landm_t = jnp.transpose(landm_data, (0, 2, 1))
    pri_t = jnp.transpose(priors, (1, 0))

    v, pos, sums = pl.pallas_call(
        _phase1_body,
        grid=(B,),
        in_specs=[
            pl.BlockSpec((1, _C, P), lambda b: (b, 0, 0)),
            pl.BlockSpec((1, 4, P), lambda b: (b, 0, 0)),
            pl.BlockSpec((1, 10, P), lambda b: (b, 0, 0)),
            pl.BlockSpec((4, P), lambda b: (0, 0)),
            pl.BlockSpec((1, _G, 15), lambda b: (b, 0, 0)),
        ],
        out_specs=[
            pl.BlockSpec((1, 1, P), lambda b: (b, 0, 0)),
            pl.BlockSpec((1, 1, P), lambda b: (b, 0, 0)),
            pl.BlockSpec((1, 1, 128), lambda b: (b, 0, 0)),
        ],
        out_shape=[
            jax.ShapeDtypeStruct((B, 1, P), jnp.float32),
            jax.ShapeDtypeStruct((B, 1, P), jnp.float32),
            jax.ShapeDtypeStruct((B, 1, 128), jnp.float32),
        ],
    )(cls_t, loc_t, landm_t, pri_t, targets)

    out = pl.pallas_call(
        _phase2_body,
        out_shape=jax.ShapeDtypeStruct((1, 128), jnp.float32),
    )(v.reshape(B, P), pos.reshape(B, P), sums.reshape(B, 128))

    return out[0, 0], out[0, 1], out[0, 2]
